# Initial kernel scaffold; baseline (speedup 1.0000x reference)
import jax
import jax.numpy as jnp
from jax.experimental import pallas as pl


def kernel(emb, W1, b1, W2, b2, edge_index):
    raise NotImplementedError("write your pallas kernel here")


# speedup vs baseline: 4.8923x; 4.8923x over previous
import functools

import jax
import jax.numpy as jnp
from jax import lax
from jax.experimental import pallas as pl
from jax.experimental.pallas import tpu as pltpu
from jax.experimental.pallas import tpu_sc as plsc

N = 50000
E = 800000
F = 64

NC = 2
NS = 16

EROWS = 6272
E_PAD = EROWS * 128
QTR = N // 4
AGG_ROWS = 12544

_mesh = plsc.VectorSubcoreMesh(core_axis_name="c", subcore_axis_name="s")
_sc_params = pltpu.CompilerParams(use_tc_tiling_on_sc=False)


def _agg_body(y, src2d, dst2d, zeros64, out, srcv, dstv, dloc, rows, agg_sh, sem):
    c = lax.axis_index("c")
    s = lax.axis_index("s")
    rowbase = s * (EROWS // NS)

    for p in range(2):
        base = (2 * p + c) * QTR

        pltpu.sync_copy(zeros64, rows)
        for j in range(2):
            rs = s * 784 + jnp.minimum(j * 512, 784 - 512)
            pltpu.sync_copy(rows, agg_sh.at[pl.ds(rs, 512)])
        plsc.subcore_barrier()

        def chunk(i, carry):
            r0 = rowbase + i * 4
            pltpu.sync_copy(src2d.at[pl.ds(r0, 4)], srcv)
            pltpu.sync_copy(dst2d.at[pl.ds(r0, 4)], dstv)
            for j in range(4):
                drow = dstv.at[j]
                lrow = dloc.at[j]

                def scan(l, cc):
                    off = l * 16
                    d16 = drow[pl.ds(off, 16)]
                    loc = d16 - base
                    m = (loc >= 0) & (loc < QTR)
                    lrow[pl.ds(off, 16)] = jnp.where(m, loc, QTR)
                    return cc

                lax.fori_loop(0, 8, scan, 0)
            hs = [
                pltpu.async_copy(y.at[srcv.at[j]], rows.at[pl.ds(j * 128, 128)], sem)
                for j in range(4)
            ]
            for h in hs:
                h.wait()
            for j in range(4):
                pltpu.sync_copy(rows.at[pl.ds(j * 128, 128)], agg_sh.at[dloc.at[j]], add=True)
            return carry

        lax.fori_loop(0, 98, chunk, 0)
        plsc.subcore_barrier()

        for j in range(2):
            k = s + NS * j

            @pl.when(k < 25)
            def _():
                rs = jnp.minimum(k * 512, QTR - 512)
                pltpu.sync_copy(agg_sh.at[pl.ds(rs, 512)], rows)
                pltpu.sync_copy(rows, out.at[pl.ds(base + rs, 512)])

        plsc.subcore_barrier()


_agg_call = functools.partial(
    pl.kernel,
    _agg_body,
    out_type=jax.ShapeDtypeStruct((N, F), jnp.float32),
    mesh=_mesh,
    compiler_params=_sc_params,
    scratch_types=[
        pltpu.VMEM((4, 128), jnp.int32),
        pltpu.VMEM((4, 128), jnp.int32),
        pltpu.VMEM((4, 128), jnp.int32),
        pltpu.VMEM((512, F), jnp.float32),
        pltpu.VMEM_SHARED((AGG_ROWS, F), jnp.float32),
        pltpu.SemaphoreType.DMA,
    ],
)()


_BR = 400
_GRID = N // _BR


def _dinv_kernel(c0_ref, dinv_ref):
    d = lax.rsqrt(c0_ref[...] + 1.0)
    dinv_ref[...] = jnp.broadcast_to(d[:, None], (N, F))


def _dinv(c0):
    return pl.pallas_call(
        _dinv_kernel,
        out_shape=jax.ShapeDtypeStruct((N, F), jnp.float32),
    )(c0)


def _dense1_kernel(dinv_ref, emb_ref, w1_ref, y_ref):
    y = jnp.dot(emb_ref[...], w1_ref[...], preferred_element_type=jnp.float32,
                precision=lax.Precision.HIGHEST)
    y_ref[...] = y * dinv_ref[...]


def _dense1(dinv, emb, W1):
    return pl.pallas_call(
        _dense1_kernel,
        grid=(_GRID,),
        in_specs=[
            pl.BlockSpec((_BR, F), lambda i: (i, 0)),
            pl.BlockSpec((_BR, F), lambda i: (i, 0)),
            pl.BlockSpec((F, F), lambda i: (0, 0)),
        ],
        out_specs=pl.BlockSpec((_BR, F), lambda i: (i, 0)),
        out_shape=jax.ShapeDtypeStruct((N, F), jnp.float32),
    )(dinv, emb, W1)


def _dense2_kernel(y_ref, agg_ref, dinv_ref, b_ref, w2_ref, y2_ref):
    dinv = dinv_ref[...]
    h = (agg_ref[...] + y_ref[...]) * dinv + b_ref[...][None, :]
    h = jnp.maximum(h, 0.0)
    y2 = jnp.dot(h, w2_ref[...], preferred_element_type=jnp.float32,
                 precision=lax.Precision.HIGHEST)
    y2_ref[...] = y2 * dinv


def _dense2(y1, agg1, dinv, b1, W2):
    return pl.pallas_call(
        _dense2_kernel,
        grid=(_GRID,),
        in_specs=[
            pl.BlockSpec((_BR, F), lambda i: (i, 0)),
            pl.BlockSpec((_BR, F), lambda i: (i, 0)),
            pl.BlockSpec((_BR, F), lambda i: (i, 0)),
            pl.BlockSpec((F,), lambda i: (0,)),
            pl.BlockSpec((F, F), lambda i: (0, 0)),
        ],
        out_specs=pl.BlockSpec((_BR, F), lambda i: (i, 0)),
        out_shape=jax.ShapeDtypeStruct((N, F), jnp.float32),
    )(y1, agg1, dinv, b1, W2)


def _dense3_kernel(y_ref, agg_ref, dinv_ref, b_ref, out_ref):
    out_ref[...] = (agg_ref[...] + y_ref[...]) * dinv_ref[...] + b_ref[...][None, :]


def _dense3(y2, agg2, dinv, b2):
    return pl.pallas_call(
        _dense3_kernel,
        grid=(_GRID,),
        in_specs=[
            pl.BlockSpec((_BR, F), lambda i: (i, 0)),
            pl.BlockSpec((_BR, F), lambda i: (i, 0)),
            pl.BlockSpec((_BR, F), lambda i: (i, 0)),
            pl.BlockSpec((F,), lambda i: (0,)),
        ],
        out_specs=pl.BlockSpec((_BR, F), lambda i: (i, 0)),
        out_shape=jax.ShapeDtypeStruct((N, F), jnp.float32),
    )(y2, agg2, dinv, b2)


def kernel(emb, W1, b1, W2, b2, edge_index):
    src = edge_index[0].astype(jnp.int32)
    dst = edge_index[1].astype(jnp.int32)
    pad = E_PAD - E
    src2d = jnp.concatenate([src, jnp.zeros((pad,), jnp.int32)]).reshape(EROWS, 128)
    dst2d = jnp.concatenate([dst, jnp.full((pad,), N, jnp.int32)]).reshape(EROWS, 128)

    zeros64 = jnp.zeros((512, F), jnp.float32)
    ones_nf = jnp.ones((N, F), jnp.float32)

    hp = _agg_call(ones_nf, src2d, dst2d, zeros64)
    dinv = _dinv(hp[:, 0])
    y1 = _dense1(dinv, emb, W1)
    agg1 = _agg_call(y1, src2d, dst2d, zeros64)
    y2 = _dense2(y1, agg1, dinv, b1, W2)
    agg2 = _agg_call(y2, src2d, dst2d, zeros64)
    return _dense3(y2, agg2, dinv, b2)


# speedup vs baseline: 6.1162x; 1.2502x over previous
import functools

import jax
import jax.numpy as jnp
from jax import lax
from jax.experimental import pallas as pl
from jax.experimental.pallas import tpu as pltpu
from jax.experimental.pallas import tpu_sc as plsc

N = 50000
E = 800000
F = 64

NC = 2
NS = 16

EROWS = 6272
E_PAD = EROWS * 128
ROWS_PT = EROWS // NS
NCHUNKS = ROWS_PT // 4

QTR = N // 4
AGG_ROWS = 12544
HALF = N // 2
DEG_ROWS = 25088
DEG_W = 8

_mesh = plsc.VectorSubcoreMesh(core_axis_name="c", subcore_axis_name="s")
_sc_params = pltpu.CompilerParams(use_tc_tiling_on_sc=False)


def _scan_dloc(dv, lv, base, rng):
    for j in range(4):
        drow = dv.at[j]
        lrow = lv.at[j]

        def scan(l, cc):
            off = l * 16
            d16 = drow[pl.ds(off, 16)]
            loc = d16 - base
            m = (loc >= 0) & (loc < rng)
            lrow[pl.ds(off, 16)] = jnp.where(m, loc, rng)
            return cc

        lax.fori_loop(0, 8, scan, 0)


def _scan_dloc8(dv, lv, base, rng):
    for j in range(8):
        drow = dv.at[j]
        lrow = lv.at[j]

        def scan(l, cc):
            off = l * 16
            d16 = drow[pl.ds(off, 16)]
            loc = d16 - base
            m = (loc >= 0) & (loc < rng)
            lrow[pl.ds(off, 16)] = jnp.where(m, loc, rng)
            return cc

        lax.fori_loop(0, 8, scan, 0)


def _deg_body(dst2d, zeros8, ones128, out, dv, dloc, zv, ov, deg_sh, sem):
    c = lax.axis_index("c")
    s = lax.axis_index("s")
    base = c * HALF
    rowbase = s * ROWS_PT

    pltpu.sync_copy(zeros8, zv)
    pltpu.sync_copy(ones128, ov)
    for j in range(4):
        rs = s * 1568 + jnp.minimum(j * 512, 1568 - 512)
        pltpu.sync_copy(zv, deg_sh.at[pl.ds(rs, 512)])
    plsc.subcore_barrier()

    def chunk(i, carry):
        pltpu.sync_copy(dst2d.at[pl.ds(rowbase + i * 4, 4)], dv)
        _scan_dloc(dv, dloc, base, HALF)
        for j in range(4):
            pltpu.sync_copy(ov, deg_sh.at[dloc.at[j]], add=True)
        return carry

    lax.fori_loop(0, NCHUNKS, chunk, 0)
    plsc.subcore_barrier()

    for j in range(4):
        k = s + NS * j

        @pl.when(k < 49)
        def _():
            rs = jnp.minimum(k * 512, HALF - 512)
            pltpu.sync_copy(deg_sh.at[pl.ds(rs, 512)], zv)
            pltpu.sync_copy(zv, out.at[pl.ds(base + rs, 512)])


_deg_call = functools.partial(
    pl.kernel,
    _deg_body,
    out_type=jax.ShapeDtypeStruct((N, DEG_W), jnp.float32),
    mesh=_mesh,
    compiler_params=_sc_params,
    scratch_types=[
        pltpu.VMEM((4, 128), jnp.int32),
        pltpu.VMEM((4, 128), jnp.int32),
        pltpu.VMEM((512, DEG_W), jnp.float32),
        pltpu.VMEM((128, DEG_W), jnp.float32),
        pltpu.VMEM_SHARED((DEG_ROWS, DEG_W), jnp.float32),
        pltpu.SemaphoreType.DMA,
    ],
)()


def _agg_body(y, src2d, dst2d, zeros64, out,
              srcv, dstv, dloc, rows, agg_sh, gsem0, gsem1, gsem2, gsem3):
    c = lax.axis_index("c")
    s = lax.axis_index("s")
    rowbase = s * ROWS_PT
    gsems = (gsem0, gsem1, gsem2, gsem3)

    for p in range(2):
        base = (2 * p + c) * QTR

        pltpu.sync_copy(zeros64, rows.at[0])
        for j in range(4):
            rs = s * 784 + jnp.minimum(j * 256, 784 - 256)
            pltpu.sync_copy(rows.at[0], agg_sh.at[pl.ds(rs, 256)])
        plsc.subcore_barrier()

        def body(i, carry):
            r0 = rowbase + i * 8
            pltpu.sync_copy(src2d.at[pl.ds(r0, 8)], srcv)
            pltpu.sync_copy(dst2d.at[pl.ds(r0, 8)], dstv)
            _scan_dloc8(dstv, dloc, base, QTR)
            hs = []
            for q in range(4):
                hs.append([
                    pltpu.async_copy(
                        y.at[srcv.at[2 * q + j]],
                        rows.at[q, pl.ds(j * 128, 128)], gsems[q])
                    for j in range(2)
                ])
            for q in range(4):
                for h in hs[q]:
                    h.wait()
                for j in range(2):
                    pltpu.sync_copy(
                        rows.at[q, pl.ds(j * 128, 128)],
                        agg_sh.at[dloc.at[2 * q + j]], add=True)
            return carry

        lax.fori_loop(0, ROWS_PT // 8, body, 0)
        plsc.subcore_barrier()

        for j in range(4):
            k = s + NS * j

            @pl.when(k < 49)
            def _():
                rs = jnp.minimum(k * 256, QTR - 256)
                pltpu.sync_copy(agg_sh.at[pl.ds(rs, 256)], rows.at[0])
                pltpu.sync_copy(rows.at[0], out.at[pl.ds(base + rs, 256)])

        plsc.subcore_barrier()


_agg_call = functools.partial(
    pl.kernel,
    _agg_body,
    out_type=jax.ShapeDtypeStruct((N, F), jnp.float32),
    mesh=_mesh,
    compiler_params=_sc_params,
    scratch_types=[
        pltpu.VMEM((8, 128), jnp.int32),
        pltpu.VMEM((8, 128), jnp.int32),
        pltpu.VMEM((8, 128), jnp.int32),
        pltpu.VMEM((4, 256, F), jnp.float32),
        pltpu.VMEM_SHARED((AGG_ROWS, F), jnp.float32),
        pltpu.SemaphoreType.DMA,
        pltpu.SemaphoreType.DMA,
        pltpu.SemaphoreType.DMA,
        pltpu.SemaphoreType.DMA,
    ],
)()


_BR = 400
_GRID = N // _BR


def _dinv_kernel(c0_ref, dinv_ref):
    d = lax.rsqrt(c0_ref[...] + 1.0)
    dinv_ref[...] = jnp.broadcast_to(d[:, None], (N, F))


def _dinv(c0):
    return pl.pallas_call(
        _dinv_kernel,
        out_shape=jax.ShapeDtypeStruct((N, F), jnp.float32),
    )(c0)


def _dense1_kernel(dinv_ref, emb_ref, w1_ref, y_ref):
    y = jnp.dot(emb_ref[...], w1_ref[...], preferred_element_type=jnp.float32,
                precision=lax.Precision.HIGHEST)
    y_ref[...] = y * dinv_ref[...]


def _dense1(dinv, emb, W1):
    return pl.pallas_call(
        _dense1_kernel,
        grid=(_GRID,),
        in_specs=[
            pl.BlockSpec((_BR, F), lambda i: (i, 0)),
            pl.BlockSpec((_BR, F), lambda i: (i, 0)),
            pl.BlockSpec((F, F), lambda i: (0, 0)),
        ],
        out_specs=pl.BlockSpec((_BR, F), lambda i: (i, 0)),
        out_shape=jax.ShapeDtypeStruct((N, F), jnp.float32),
    )(dinv, emb, W1)


def _dense2_kernel(y_ref, agg_ref, dinv_ref, b_ref, w2_ref, y2_ref):
    dinv = dinv_ref[...]
    h = (agg_ref[...] + y_ref[...]) * dinv + b_ref[...][None, :]
    h = jnp.maximum(h, 0.0)
    y2 = jnp.dot(h, w2_ref[...], preferred_element_type=jnp.float32,
                 precision=lax.Precision.HIGHEST)
    y2_ref[...] = y2 * dinv


def _dense2(y1, agg1, dinv, b1, W2):
    return pl.pallas_call(
        _dense2_kernel,
        grid=(_GRID,),
        in_specs=[
            pl.BlockSpec((_BR, F), lambda i: (i, 0)),
            pl.BlockSpec((_BR, F), lambda i: (i, 0)),
            pl.BlockSpec((_BR, F), lambda i: (i, 0)),
            pl.BlockSpec((F,), lambda i: (0,)),
            pl.BlockSpec((F, F), lambda i: (0, 0)),
        ],
        out_specs=pl.BlockSpec((_BR, F), lambda i: (i, 0)),
        out_shape=jax.ShapeDtypeStruct((N, F), jnp.float32),
    )(y1, agg1, dinv, b1, W2)


def _dense3_kernel(y_ref, agg_ref, dinv_ref, b_ref, out_ref):
    out_ref[...] = (agg_ref[...] + y_ref[...]) * dinv_ref[...] + b_ref[...][None, :]


def _dense3(y2, agg2, dinv, b2):
    return pl.pallas_call(
        _dense3_kernel,
        grid=(_GRID,),
        in_specs=[
            pl.BlockSpec((_BR, F), lambda i: (i, 0)),
            pl.BlockSpec((_BR, F), lambda i: (i, 0)),
            pl.BlockSpec((_BR, F), lambda i: (i, 0)),
            pl.BlockSpec((F,), lambda i: (0,)),
        ],
        out_specs=pl.BlockSpec((_BR, F), lambda i: (i, 0)),
        out_shape=jax.ShapeDtypeStruct((N, F), jnp.float32),
    )(y2, agg2, dinv, b2)


def kernel(emb, W1, b1, W2, b2, edge_index):
    src = edge_index[0].astype(jnp.int32)
    dst = edge_index[1].astype(jnp.int32)
    pad = E_PAD - E
    src2d = jnp.concatenate([src, jnp.zeros((pad,), jnp.int32)]).reshape(EROWS, 128)
    dst2d = jnp.concatenate([dst, jnp.full((pad,), N, jnp.int32)]).reshape(EROWS, 128)

    zeros8 = jnp.zeros((512, DEG_W), jnp.float32)
    ones128 = jnp.ones((128, DEG_W), jnp.float32)
    zeros64 = jnp.zeros((256, F), jnp.float32)

    hp = _deg_call(dst2d, zeros8, ones128)
    dinv = _dinv(hp[:, 0])
    y1 = _dense1(dinv, emb, W1)
    agg1 = _agg_call(y1, src2d, dst2d, zeros64)
    y2 = _dense2(y1, agg1, dinv, b1, W2)
    agg2 = _agg_call(y2, src2d, dst2d, zeros64)
    return _dense3(y2, agg2, dinv, b2)
